# baseline (speedup 1.0000x reference)
import jax
import jax.numpy as jnp
from jax import lax
from jax.experimental import pallas as pl
from jax.experimental.pallas import tpu as pltpu
from jax.experimental.pallas import tpu_sc as plsc

_NCORE = 2
_NSUB = 16
_NW = _NCORE * _NSUB
_CHUNK = 128
_TW = 80
_BLK = 1024


def _gmat(fw, h):
    c = fw // h
    r = lax.broadcasted_iota(jnp.int32, (fw, h), 0) // c
    q = lax.broadcasted_iota(jnp.int32, (fw, h), 1)
    return (r == q).astype(jnp.float32)


def _smat(h, fw):
    c = fw // h
    r = lax.broadcasted_iota(jnp.int32, (h, fw), 0)
    q = lax.broadcasted_iota(jnp.int32, (h, fw), 1) // c
    return (r == q).astype(jnp.float32)


def _embed1(x, W, attl, attr, H, dummy):
    N, F = x.shape
    FW = W.shape[1]
    pad = _TW - FW - 2 * H

    def body(x_ref, w_ref, al_ref, ar_ref, o_ref):
        h = jnp.dot(x_ref[...], w_ref[...], preferred_element_type=jnp.float32)
        G = _gmat(FW, H)
        al = jnp.dot(h * al_ref[...], G, preferred_element_type=jnp.float32)
        ar = jnp.dot(h * ar_ref[...], G, preferred_element_type=jnp.float32)
        rid = (lax.broadcasted_iota(jnp.int32, (_BLK, 1), 0)
               + pl.program_id(0) * _BLK)
        al = jnp.where(rid == dummy, jnp.float32(-3e38), al)
        parts = [h, al, ar]
        if pad:
            parts.append(jnp.zeros((h.shape[0], pad), jnp.float32))
        o_ref[...] = jnp.concatenate(parts, axis=1)

    return pl.pallas_call(
        body,
        grid=(N // _BLK,),
        in_specs=[
            pl.BlockSpec((_BLK, F), lambda i: (i, 0)),
            pl.BlockSpec((F, FW), lambda i: (0, 0)),
            pl.BlockSpec((1, FW), lambda i: (0, 0)),
            pl.BlockSpec((1, FW), lambda i: (0, 0)),
        ],
        out_specs=pl.BlockSpec((_BLK, _TW), lambda i: (i, 0)),
        out_shape=jax.ShapeDtypeStruct((N, _TW), jnp.float32),
    )(x, W, attl, attr)


def _mid(acc, b1, W2, attl2, attr2, H, dummy):
    _, N, AW = acc.shape
    FW1 = W2.shape[0]
    FW2 = W2.shape[1]
    pad = _TW - FW2 - 2 * H

    def body(a_ref, b_ref, w_ref, al_ref, ar_ref, o_ref):
        p = a_ref[0] + a_ref[1]
        feats = p[:, :FW1]
        den = p[:, FW1:FW1 + H] + 1e-16
        db = jnp.dot(den, _smat(H, FW1), preferred_element_type=jnp.float32)
        x1 = feats / db + b_ref[...]
        x1 = jnp.where(x1 > 0, x1, jnp.exp(jnp.minimum(x1, 0.0)) - 1.0)
        h2 = jnp.dot(x1, w_ref[...], preferred_element_type=jnp.float32)
        G = _gmat(FW2, H)
        al = jnp.dot(h2 * al_ref[...], G, preferred_element_type=jnp.float32)
        ar = jnp.dot(h2 * ar_ref[...], G, preferred_element_type=jnp.float32)
        rid = (lax.broadcasted_iota(jnp.int32, (_BLK, 1), 0)
               + pl.program_id(0) * _BLK)
        al = jnp.where(rid == dummy, jnp.float32(-3e38), al)
        parts = [h2, al, ar]
        if pad:
            parts.append(jnp.zeros((h2.shape[0], pad), jnp.float32))
        o_ref[...] = jnp.concatenate(parts, axis=1)

    return pl.pallas_call(
        body,
        grid=(N // _BLK,),
        in_specs=[
            pl.BlockSpec((_NCORE, _BLK, AW), lambda i: (0, i, 0)),
            pl.BlockSpec((1, FW1), lambda i: (0, 0)),
            pl.BlockSpec((FW1, FW2), lambda i: (0, 0)),
            pl.BlockSpec((1, FW2), lambda i: (0, 0)),
            pl.BlockSpec((1, FW2), lambda i: (0, 0)),
        ],
        out_specs=pl.BlockSpec((_BLK, _TW), lambda i: (i, 0)),
        out_shape=jax.ShapeDtypeStruct((N, _TW), jnp.float32),
    )(acc, b1, W2, attl2, attr2)


def _final(acc, b2, H, C):
    _, N, AW = acc.shape
    FW = H * C
    NC = b2.shape[1]

    def body(a_ref, b_ref, o_ref):
        p = a_ref[0] + a_ref[1]
        feats = p[:, :FW]
        den = p[:, FW:FW + H] + 1e-16
        db = jnp.dot(den, _smat(H, FW), preferred_element_type=jnp.float32)
        xo = feats / db
        r = lax.broadcasted_iota(jnp.int32, (FW, C), 0) % C
        q = lax.broadcasted_iota(jnp.int32, (FW, C), 1)
        M = (r == q).astype(jnp.float32)
        z = jnp.dot(xo, M, preferred_element_type=jnp.float32) * (1.0 / H)
        z = z + b_ref[...]
        m = jnp.max(z, axis=1, keepdims=True)
        lse = jnp.log(jnp.sum(jnp.exp(z - m), axis=1, keepdims=True))
        o_ref[...] = z - m - lse

    return pl.pallas_call(
        body,
        grid=(N // _BLK,),
        in_specs=[
            pl.BlockSpec((_NCORE, _BLK, AW), lambda i: (0, i, 0)),
            pl.BlockSpec((1, NC), lambda i: (0, 0)),
        ],
        out_specs=pl.BlockSpec((_BLK, NC), lambda i: (i, 0)),
        out_shape=jax.ShapeDtypeStruct((N, NC), jnp.float32),
    )(acc, b2)


def _pack_table(T, H, C):
    Npad = T.shape[0]
    FW = H * C
    f = T[:, :FW].reshape(Npad, H, C)
    if C < 8:
        f = jnp.pad(f, ((0, 0), (0, 0), (0, 8 - C)))
    f = f.reshape(Npad, H * 8).astype(jnp.bfloat16)
    u = lax.bitcast_convert_type(f, jnp.uint16).astype(jnp.uint32)
    w = u[:, 0::2] | (u[:, 1::2] << 16)
    alw = lax.bitcast_convert_type(T[:, FW:FW + H], jnp.uint32)
    arw = lax.bitcast_convert_type(T[:, FW + H:FW + 2 * H], jnp.uint32)
    packed = jnp.concatenate([w, alw, arw], axis=1)
    return lax.bitcast_convert_type(packed, jnp.int32)


def _edge_accumulate(T, src, dst, zeros, H, C):
    Npad = T.shape[0]
    TWP = T.shape[1]
    FW = H * C
    accw = FW + H
    if (accw // 8) % 2 == 0:
        accw += 8
    Epad = src.shape[0]
    cpw = Epad // (_NW * _CHUNK)
    rps = Npad // _NSUB
    mesh = plsc.VectorSubcoreMesh(
        core_axis_name="c", subcore_axis_name="s",
        num_cores=_NCORE, num_subcores=_NSUB)

    def body(t_ref, src_ref, dst_ref, z_ref, out_ref, *scr):
        srcb = scr[0:6]
        dstb = scr[6:12]
        tj = scr[12:14]
        ti = scr[14:16]
        wx = scr[16:18]
        acc = scr[18]
        semx = scr[19:25]
        semj = scr[25:27]
        semi = scr[27:29]
        sems = scr[29:31]
        cid = lax.axis_index("c")
        sid = lax.axis_index("s")
        wid = cid * _NSUB + sid
        s_lo = sid * rps
        pltpu.sync_copy(z_ref.at[pl.ds(s_lo, rps)], acc.at[pl.ds(s_lo, rps)])
        lane = lax.broadcasted_iota(jnp.int32, (16,), 0)
        if FW + H < accw:
            zero16 = jnp.zeros((16,), jnp.float32)
            for wxr in wx:
                for g in range(_CHUNK // 16):
                    rows = g * 16 + lane
                    for cc in range(FW + H, accw):
                        plsc.store_scatter(
                            wxr, [rows, jnp.full((16,), cc, jnp.int32)],
                            zero16)
        plsc.subcore_barrier()

        def fire_idx(k, s6):
            off = (k * _NW + wid) * _CHUNK
            pltpu.async_copy(src_ref.at[pl.ds(off, _CHUNK)], srcb[s6],
                             semx[s6])
            pltpu.async_copy(dst_ref.at[pl.ds(off, _CHUNK)], dstb[s6],
                             semx[s6])

        def wait_idx(k, s6):
            off = (k * _NW + wid) * _CHUNK
            pltpu.make_async_copy(src_ref.at[pl.ds(off, _CHUNK)], srcb[s6],
                                  semx[s6]).wait()
            pltpu.make_async_copy(dst_ref.at[pl.ds(off, _CHUNK)], dstb[s6],
                                  semx[s6]).wait()

        def fire_gathers(s3, s6):
            pltpu.async_copy(t_ref.at[srcb[s6]], tj[s3], semj[s3])
            pltpu.async_copy(t_ref.at[dstb[s6]], ti[s3], semi[s3])

        def wait_gathers(s3, s6):
            pltpu.make_async_copy(t_ref.at[srcb[s6]], tj[s3], semj[s3]).wait()
            pltpu.make_async_copy(t_ref.at[dstb[s6]], ti[s3], semi[s3]).wait()

        def fire_scatter(s3, s6):
            pltpu.async_copy(wx[s3], acc.at[dstb[s6]], sems[s3], add=True)

        def wait_scatter(s3, s6):
            pltpu.make_async_copy(wx[s3], acc.at[dstb[s6]], sems[s3]).wait()

        def compute(s3):
            tjr, tir, wxr = tj[s3], ti[s3], wx[s3]

            himask = jnp.int32(-65536)

            def unpack(v):
                lo = plsc.bitcast(jnp.left_shift(v, 16), jnp.float32)
                hi = plsc.bitcast(jnp.bitwise_and(v, himask), jnp.float32)
                return lo, hi

            def group_body(g, carry2):
                rows = g * 16 + lane
                for hb in range(0, H, 2):
                    hs = range(hb, min(hb + 2, H))
                    kept, logit, att = {}, {}, {}
                    for h in hs:
                        ps, cols = [], []
                        for k in range(4):
                            col = jnp.full((16,), 4 * h + k, jnp.int32)
                            wj = plsc.load_gather(tjr, [rows, col])
                            wi = plsc.load_gather(tir, [rows, col])
                            jlo, jhi = unpack(wj)
                            ilo, ihi = unpack(wi)
                            cols += [jlo, jhi]
                            ps.append(jlo * ilo + jhi * ihi)
                        while len(ps) > 1:
                            nxt = [ps[i] + ps[i + 1]
                                   for i in range(0, len(ps) - 1, 2)]
                            if len(ps) % 2:
                                nxt.append(ps[-1])
                            ps = nxt
                        alj = plsc.bitcast(plsc.load_gather(
                            tjr, [rows, jnp.full((16,), 4 * H + h, jnp.int32)]),
                            jnp.float32)
                        ari = plsc.bitcast(plsc.load_gather(
                            tir, [rows, jnp.full((16,), 5 * H + h, jnp.int32)]),
                            jnp.float32)
                        kept[h], logit[h], att[h] = cols, ps[0], alj + ari
                    ws = {}
                    for h in hs:
                        sig = 1.0 / (1.0 + jnp.exp(-logit[h]))
                        a = att[h] * sig
                        a = jnp.where(a >= 0, a, 0.2 * a)
                        ws[h] = jnp.exp(a)
                    for h in hs:
                        plsc.store_scatter(
                            wxr, [rows, jnp.full((16,), FW + h, jnp.int32)],
                            ws[h])
                        for c in range(C):
                            col = jnp.full((16,), h * C + c, jnp.int32)
                            plsc.store_scatter(wxr, [rows, col],
                                               ws[h] * kept[h][c])
                return carry2

            lax.fori_loop(0, _CHUNK // 16, group_body, 0)

        fire_idx(0, 0)
        fire_idx(1, 1)
        wait_idx(0, 0)
        fire_gathers(0, 0)

        M = cpw // 6

        def six_body(m, carry):
            k0 = 6 * m
            for p in range(6):
                k = k0 + p
                s2, s6 = p % 2, p
                n2, n6 = (p + 1) % 2, (p + 1) % 6
                wait_gathers(s2, s6)
                if p < 2:
                    @pl.when(m > 0)
                    def _():
                        wait_scatter(s2, (p - 2) % 6)
                else:
                    wait_scatter(s2, (p - 2) % 6)
                if p < 5:
                    wait_idx(k + 1, n6)
                    fire_gathers(n2, n6)
                    if p == 4:
                        @pl.when(m < M - 1)
                        def _():
                            fire_idx(k + 2, (p + 2) % 6)
                    else:
                        fire_idx(k + 2, (p + 2) % 6)
                else:
                    @pl.when(m < M - 1)
                    def _():
                        wait_idx(k + 1, n6)
                        fire_gathers(n2, n6)
                        fire_idx(k + 2, (p + 2) % 6)
                compute(s2)
                fire_scatter(s2, s6)
            return carry

        lax.fori_loop(0, M, six_body, 0)
        wait_scatter((cpw - 2) % 2, (cpw - 2) % 6)
        wait_scatter((cpw - 1) % 2, (cpw - 1) % 6)
        plsc.subcore_barrier()
        pltpu.sync_copy(acc.at[pl.ds(s_lo, rps)],
                        out_ref.at[cid, pl.ds(s_lo, rps)])

    fn = pl.kernel(
        body,
        out_type=jax.ShapeDtypeStruct((_NCORE, Npad, accw), jnp.float32),
        mesh=mesh,
        compiler_params=pltpu.CompilerParams(
            needs_layout_passes=False, use_tc_tiling_on_sc=False),
        scratch_types=(
            [pltpu.VMEM((_CHUNK,), jnp.int32) for _ in range(12)]
            + [pltpu.VMEM((_CHUNK, TWP), jnp.int32) for _ in range(4)]
            + [pltpu.VMEM((_CHUNK, accw), jnp.float32) for _ in range(2)]
            + [pltpu.VMEM_SHARED((Npad, accw), jnp.float32)]
            + [pltpu.SemaphoreType.DMA for _ in range(12)]
        ),
    )
    return fn(T, src, dst, zeros)


def kernel(x, edge_index, W1, att_l1, att_r1, b1, W2, att_l2, att_r2, b2):
    N, F = x.shape
    E = edge_index.shape[1]
    H, C1 = att_l1.shape[1], att_l1.shape[2]
    C2 = att_l2.shape[2]
    Etot = E + N
    Npad = -(-N // 2048) * 2048
    dummy = Npad - 1

    loop = jnp.arange(N, dtype=jnp.int32)
    e_src, e_dst = edge_index[0], edge_index[1]
    e_src = jnp.where(e_src == e_dst, jnp.int32(dummy), e_src)
    src = jnp.concatenate([e_src, loop])
    dst = jnp.concatenate([e_dst, loop])
    cpw = -(-Etot // (_NW * _CHUNK))
    cpw = -(-cpw // 6) * 6
    Epad = cpw * _NW * _CHUNK
    pad = Epad - Etot
    if pad:
        src = jnp.concatenate([src, jnp.full((pad,), dummy, jnp.int32)])
        dst = jnp.concatenate([dst, jnp.arange(pad, dtype=jnp.int32) % N])

    def _accw(c):
        w = H * c + H
        return w + 8 if (w // 8) % 2 == 0 else w

    xp = jnp.pad(x, ((0, Npad - N), (0, 0)))
    zeros1 = jnp.zeros((Npad, _accw(C1)), jnp.float32)
    zeros2 = jnp.zeros((Npad, _accw(C2)), jnp.float32)

    T1 = _embed1(xp, W1, att_l1.reshape(1, -1), att_r1.reshape(1, -1), H,
                 dummy)
    acc1 = _edge_accumulate(_pack_table(T1, H, C1), src, dst, zeros1, H, C1)
    T2 = _mid(acc1, b1.reshape(1, -1), W2,
              att_l2.reshape(1, -1), att_r2.reshape(1, -1), H, dummy)
    acc2 = _edge_accumulate(_pack_table(T2, H, C2), src, dst, zeros2, H, C2)
    logp = _final(acc2, b2.reshape(1, -1), H, C2)
    return (logp[:N], jnp.float32(0.0))


# speedup vs baseline: 1.3559x; 1.3559x over previous
import jax
import jax.numpy as jnp
from jax import lax
from jax.experimental import pallas as pl
from jax.experimental.pallas import tpu as pltpu
from jax.experimental.pallas import tpu_sc as plsc

_NCORE = 2
_NSUB = 16
_NW = _NCORE * _NSUB
_CHUNK = 128
_TW = 80
_BLK = 1024


def _gmat(fw, h):
    c = fw // h
    r = lax.broadcasted_iota(jnp.int32, (fw, h), 0) // c
    q = lax.broadcasted_iota(jnp.int32, (fw, h), 1)
    return (r == q).astype(jnp.float32)


def _smat(h, fw):
    c = fw // h
    r = lax.broadcasted_iota(jnp.int32, (h, fw), 0)
    q = lax.broadcasted_iota(jnp.int32, (h, fw), 1) // c
    return (r == q).astype(jnp.float32)


def _embed1(x, W, attl, attr, H, dummy):
    N, F = x.shape
    FW = W.shape[1]
    pad = _TW - FW - 2 * H

    def body(x_ref, w_ref, al_ref, ar_ref, o_ref):
        h = jnp.dot(x_ref[...], w_ref[...], preferred_element_type=jnp.float32)
        G = _gmat(FW, H)
        al = jnp.dot(h * al_ref[...], G, preferred_element_type=jnp.float32)
        ar = jnp.dot(h * ar_ref[...], G, preferred_element_type=jnp.float32)
        rid = (lax.broadcasted_iota(jnp.int32, (_BLK, 1), 0)
               + pl.program_id(0) * _BLK)
        al = jnp.where(rid == dummy, jnp.float32(-3e38), al)
        parts = [h, al, ar]
        if pad:
            parts.append(jnp.zeros((h.shape[0], pad), jnp.float32))
        o_ref[...] = jnp.concatenate(parts, axis=1)

    return pl.pallas_call(
        body,
        grid=(N // _BLK,),
        in_specs=[
            pl.BlockSpec((_BLK, F), lambda i: (i, 0)),
            pl.BlockSpec((F, FW), lambda i: (0, 0)),
            pl.BlockSpec((1, FW), lambda i: (0, 0)),
            pl.BlockSpec((1, FW), lambda i: (0, 0)),
        ],
        out_specs=pl.BlockSpec((_BLK, _TW), lambda i: (i, 0)),
        out_shape=jax.ShapeDtypeStruct((N, _TW), jnp.float32),
    )(x, W, attl, attr)


def _mid(acc, b1, W2, attl2, attr2, H, dummy):
    _, N, AW = acc.shape
    FW1 = W2.shape[0]
    FW2 = W2.shape[1]
    pad = _TW - FW2 - 2 * H

    def body(a_ref, b_ref, w_ref, al_ref, ar_ref, o_ref):
        p = a_ref[0] + a_ref[1]
        feats = p[:, :FW1]
        den = p[:, FW1:FW1 + H] + 1e-16
        db = jnp.dot(den, _smat(H, FW1), preferred_element_type=jnp.float32)
        x1 = feats / db + b_ref[...]
        x1 = jnp.where(x1 > 0, x1, jnp.exp(jnp.minimum(x1, 0.0)) - 1.0)
        h2 = jnp.dot(x1, w_ref[...], preferred_element_type=jnp.float32)
        G = _gmat(FW2, H)
        al = jnp.dot(h2 * al_ref[...], G, preferred_element_type=jnp.float32)
        ar = jnp.dot(h2 * ar_ref[...], G, preferred_element_type=jnp.float32)
        rid = (lax.broadcasted_iota(jnp.int32, (_BLK, 1), 0)
               + pl.program_id(0) * _BLK)
        al = jnp.where(rid == dummy, jnp.float32(-3e38), al)
        parts = [h2, al, ar]
        if pad:
            parts.append(jnp.zeros((h2.shape[0], pad), jnp.float32))
        o_ref[...] = jnp.concatenate(parts, axis=1)

    return pl.pallas_call(
        body,
        grid=(N // _BLK,),
        in_specs=[
            pl.BlockSpec((_NCORE, _BLK, AW), lambda i: (0, i, 0)),
            pl.BlockSpec((1, FW1), lambda i: (0, 0)),
            pl.BlockSpec((FW1, FW2), lambda i: (0, 0)),
            pl.BlockSpec((1, FW2), lambda i: (0, 0)),
            pl.BlockSpec((1, FW2), lambda i: (0, 0)),
        ],
        out_specs=pl.BlockSpec((_BLK, _TW), lambda i: (i, 0)),
        out_shape=jax.ShapeDtypeStruct((N, _TW), jnp.float32),
    )(acc, b1, W2, attl2, attr2)


def _final(acc, b2, H, C):
    _, N, AW = acc.shape
    FW = H * C
    NC = b2.shape[1]

    def body(a_ref, b_ref, o_ref):
        p = a_ref[0] + a_ref[1]
        feats = p[:, :FW]
        den = p[:, FW:FW + H] + 1e-16
        db = jnp.dot(den, _smat(H, FW), preferred_element_type=jnp.float32)
        xo = feats / db
        r = lax.broadcasted_iota(jnp.int32, (FW, C), 0) % C
        q = lax.broadcasted_iota(jnp.int32, (FW, C), 1)
        M = (r == q).astype(jnp.float32)
        z = jnp.dot(xo, M, preferred_element_type=jnp.float32) * (1.0 / H)
        z = z + b_ref[...]
        m = jnp.max(z, axis=1, keepdims=True)
        lse = jnp.log(jnp.sum(jnp.exp(z - m), axis=1, keepdims=True))
        o_ref[...] = z - m - lse

    return pl.pallas_call(
        body,
        grid=(N // _BLK,),
        in_specs=[
            pl.BlockSpec((_NCORE, _BLK, AW), lambda i: (0, i, 0)),
            pl.BlockSpec((1, NC), lambda i: (0, 0)),
        ],
        out_specs=pl.BlockSpec((_BLK, NC), lambda i: (i, 0)),
        out_shape=jax.ShapeDtypeStruct((N, NC), jnp.float32),
    )(acc, b2)


def _pack_table(T, H, C):
    Npad = T.shape[0]
    FW = H * C
    f = T[:, :FW].reshape(Npad, H, C)
    if C < 8:
        f = jnp.pad(f, ((0, 0), (0, 0), (0, 8 - C)))
    f = f.reshape(Npad, H * 8).astype(jnp.bfloat16)
    u = lax.bitcast_convert_type(f, jnp.uint16).astype(jnp.uint32)
    w = u[:, 0::2] | (u[:, 1::2] << 16)
    alw = lax.bitcast_convert_type(T[:, FW:FW + H], jnp.uint32)
    arw = lax.bitcast_convert_type(T[:, FW + H:FW + 2 * H], jnp.uint32)
    packed = jnp.concatenate([w, alw, arw], axis=1)
    return lax.bitcast_convert_type(packed, jnp.int32)


def _edge_accumulate(T, src, dst, zeros, H, C):
    Npad = T.shape[0]
    TWP = T.shape[1]
    FW = H * C
    accw = FW + H
    if (accw // 8) % 2 == 0:
        accw += 8
    Epad = src.shape[0]
    cpw = Epad // (_NW * _CHUNK)
    rps = Npad // _NSUB
    mesh = plsc.VectorSubcoreMesh(
        core_axis_name="c", subcore_axis_name="s",
        num_cores=_NCORE, num_subcores=_NSUB)

    def body(t_ref, src_ref, dst_ref, z_ref, out_ref, *scr):
        srcb = scr[0:6]
        dstb = scr[6:12]
        tj = scr[12:14]
        ti = scr[14:16]
        wx = scr[16:18]
        acc = scr[18]
        semx = scr[19:25]
        semj = scr[25:27]
        semi = scr[27:29]
        sems = scr[29:31]
        cid = lax.axis_index("c")
        sid = lax.axis_index("s")
        wid = cid * _NSUB + sid
        s_lo = sid * rps
        pltpu.sync_copy(z_ref.at[pl.ds(s_lo, rps)], acc.at[pl.ds(s_lo, rps)])
        lane = lax.broadcasted_iota(jnp.int32, (16,), 0)
        if FW + H < accw:
            zero16 = jnp.zeros((16,), jnp.float32)
            for wxr in wx:
                for g in range(_CHUNK // 16):
                    rows = g * 16 + lane
                    for cc in range(FW + H, accw):
                        plsc.store_scatter(
                            wxr, [rows, jnp.full((16,), cc, jnp.int32)],
                            zero16)
        plsc.subcore_barrier()

        base = wid * (cpw * _CHUNK)

        def fire_idx(k, s6):
            off = base + k * _CHUNK
            pltpu.async_copy(src_ref.at[pl.ds(off, _CHUNK)], srcb[s6],
                             semx[s6])
            pltpu.async_copy(dst_ref.at[pl.ds(off, _CHUNK)], dstb[s6],
                             semx[s6])

        def wait_idx(k, s6):
            off = base + k * _CHUNK
            pltpu.make_async_copy(src_ref.at[pl.ds(off, _CHUNK)], srcb[s6],
                                  semx[s6]).wait()
            pltpu.make_async_copy(dst_ref.at[pl.ds(off, _CHUNK)], dstb[s6],
                                  semx[s6]).wait()

        def fire_gathers(s3, s6):
            pltpu.async_copy(t_ref.at[srcb[s6]], tj[s3], semj[s3])
            pltpu.async_copy(t_ref.at[dstb[s6]], ti[s3], semi[s3])

        def wait_gathers(s3, s6):
            pltpu.make_async_copy(t_ref.at[srcb[s6]], tj[s3], semj[s3]).wait()
            pltpu.make_async_copy(t_ref.at[dstb[s6]], ti[s3], semi[s3]).wait()

        def fire_scatter(s3, s6):
            pltpu.async_copy(wx[s3], acc.at[dstb[s6]], sems[s3], add=True)

        def wait_scatter(s3, s6):
            pltpu.make_async_copy(wx[s3], acc.at[dstb[s6]], sems[s3]).wait()

        def compute(s3):
            tjr, tir, wxr = tj[s3], ti[s3], wx[s3]

            himask = jnp.int32(-65536)

            def unpack(v):
                lo = plsc.bitcast(jnp.left_shift(v, 16), jnp.float32)
                hi = plsc.bitcast(jnp.bitwise_and(v, himask), jnp.float32)
                return lo, hi

            def group_body(g, carry2):
                rows = g * 16 + lane
                for hb in range(0, H, 2):
                    hs = range(hb, min(hb + 2, H))
                    kept, logit, att = {}, {}, {}
                    for h in hs:
                        ps, cols = [], []
                        for k in range(4):
                            col = jnp.full((16,), 4 * h + k, jnp.int32)
                            wj = plsc.load_gather(tjr, [rows, col])
                            wi = plsc.load_gather(tir, [rows, col])
                            jlo, jhi = unpack(wj)
                            ilo, ihi = unpack(wi)
                            cols += [jlo, jhi]
                            ps.append(jlo * ilo + jhi * ihi)
                        while len(ps) > 1:
                            nxt = [ps[i] + ps[i + 1]
                                   for i in range(0, len(ps) - 1, 2)]
                            if len(ps) % 2:
                                nxt.append(ps[-1])
                            ps = nxt
                        alj = plsc.bitcast(plsc.load_gather(
                            tjr, [rows, jnp.full((16,), 4 * H + h, jnp.int32)]),
                            jnp.float32)
                        ari = plsc.bitcast(plsc.load_gather(
                            tir, [rows, jnp.full((16,), 5 * H + h, jnp.int32)]),
                            jnp.float32)
                        kept[h], logit[h], att[h] = cols, ps[0], alj + ari
                    ws = {}
                    for h in hs:
                        sig = 1.0 / (1.0 + jnp.exp(-logit[h]))
                        a = att[h] * sig
                        a = jnp.where(a >= 0, a, 0.2 * a)
                        ws[h] = jnp.exp(a)
                    for h in hs:
                        plsc.store_scatter(
                            wxr, [rows, jnp.full((16,), FW + h, jnp.int32)],
                            ws[h])
                        for c in range(C):
                            col = jnp.full((16,), h * C + c, jnp.int32)
                            plsc.store_scatter(wxr, [rows, col],
                                               ws[h] * kept[h][c])
                return carry2

            lax.fori_loop(0, _CHUNK // 16, group_body, 0)

        fire_idx(0, 0)
        fire_idx(1, 1)
        wait_idx(0, 0)
        fire_gathers(0, 0)

        M = cpw // 6

        def six_body(m, carry):
            k0 = 6 * m
            for p in range(6):
                k = k0 + p
                s2, s6 = p % 2, p
                n2, n6 = (p + 1) % 2, (p + 1) % 6
                wait_gathers(s2, s6)
                if p < 2:
                    @pl.when(m > 0)
                    def _():
                        wait_scatter(s2, (p - 2) % 6)
                else:
                    wait_scatter(s2, (p - 2) % 6)
                if p < 5:
                    wait_idx(k + 1, n6)
                    fire_gathers(n2, n6)
                    if p == 4:
                        @pl.when(m < M - 1)
                        def _():
                            fire_idx(k + 2, (p + 2) % 6)
                    else:
                        fire_idx(k + 2, (p + 2) % 6)
                else:
                    @pl.when(m < M - 1)
                    def _():
                        wait_idx(k + 1, n6)
                        fire_gathers(n2, n6)
                        fire_idx(k + 2, (p + 2) % 6)
                compute(s2)
                fire_scatter(s2, s6)
            return carry

        lax.fori_loop(0, M, six_body, 0)
        wait_scatter((cpw - 2) % 2, (cpw - 2) % 6)
        wait_scatter((cpw - 1) % 2, (cpw - 1) % 6)
        plsc.subcore_barrier()
        pltpu.sync_copy(acc.at[pl.ds(s_lo, rps)],
                        out_ref.at[cid, pl.ds(s_lo, rps)])

    fn = pl.kernel(
        body,
        out_type=jax.ShapeDtypeStruct((_NCORE, Npad, accw), jnp.float32),
        mesh=mesh,
        compiler_params=pltpu.CompilerParams(
            needs_layout_passes=False, use_tc_tiling_on_sc=False),
        scratch_types=(
            [pltpu.VMEM((_CHUNK,), jnp.int32) for _ in range(12)]
            + [pltpu.VMEM((_CHUNK, TWP), jnp.int32) for _ in range(4)]
            + [pltpu.VMEM((_CHUNK, accw), jnp.float32) for _ in range(2)]
            + [pltpu.VMEM_SHARED((Npad, accw), jnp.float32)]
            + [pltpu.SemaphoreType.DMA for _ in range(12)]
        ),
    )
    return fn(T, src, dst, zeros)


def kernel(x, edge_index, W1, att_l1, att_r1, b1, W2, att_l2, att_r2, b2):
    N, F = x.shape
    E = edge_index.shape[1]
    H, C1 = att_l1.shape[1], att_l1.shape[2]
    C2 = att_l2.shape[2]
    Etot = E + N
    Npad = -(-N // 2048) * 2048
    dummy = Npad - 1

    loop = jnp.arange(N, dtype=jnp.int32)
    e_src, e_dst = edge_index[0], edge_index[1]
    e_src = jnp.where(e_src == e_dst, jnp.int32(dummy), e_src)
    src = jnp.concatenate([e_src, loop])
    dst = jnp.concatenate([e_dst, loop])
    cpw = -(-Etot // (_NW * _CHUNK))
    cpw = -(-cpw // 6) * 6
    Epad = cpw * _NW * _CHUNK
    pad = Epad - Etot
    if pad:
        src = jnp.concatenate([src, jnp.full((pad,), dummy, jnp.int32)])
        dst = jnp.concatenate([dst, jnp.arange(pad, dtype=jnp.int32) % N])

    def _accw(c):
        w = H * c + H
        return w + 8 if (w // 8) % 2 == 0 else w

    xp = jnp.pad(x, ((0, Npad - N), (0, 0)))
    zeros1 = jnp.zeros((Npad, _accw(C1)), jnp.float32)
    zeros2 = jnp.zeros((Npad, _accw(C2)), jnp.float32)

    T1 = _embed1(xp, W1, att_l1.reshape(1, -1), att_r1.reshape(1, -1), H,
                 dummy)
    acc1 = _edge_accumulate(_pack_table(T1, H, C1), src, dst, zeros1, H, C1)
    T2 = _mid(acc1, b1.reshape(1, -1), W2,
              att_l2.reshape(1, -1), att_r2.reshape(1, -1), H, dummy)
    acc2 = _edge_accumulate(_pack_table(T2, H, C2), src, dst, zeros2, H, C2)
    logp = _final(acc2, b2.reshape(1, -1), H, C2)
    return (logp[:N], jnp.float32(0.0))
